# Initial kernel scaffold; baseline (speedup 1.0000x reference)
import jax
import jax.numpy as jnp
from jax.experimental import pallas as pl


def kernel(ent, att, W_att):
    raise NotImplementedError("write your pallas kernel here")


# speedup vs baseline: 2.0268x; 2.0268x over previous
import functools

import jax
import jax.numpy as jnp
from jax import lax
from jax.experimental import pallas as pl
from jax.experimental.pallas import tpu as pltpu
from jax.experimental.pallas import tpu_sc as plsc

D = 32
DD = D * D
B = 16384
L = 16
NC, NS = 2, 16
NW = NC * NS
BPW = B // NW
K = 32
NCHUNK = BPW // K


def _sc_call(ent, att3, W_att):
    mesh = plsc.VectorSubcoreMesh(core_axis_name="c", subcore_axis_name="s")

    @functools.partial(
        pl.kernel,
        mesh=mesh,
        out_type=jax.ShapeDtypeStruct((B, D), jnp.float32),
        scratch_types=[
            pltpu.VMEM((NCHUNK, K), jnp.int32),
            pltpu.VMEM((2, K, D), jnp.float32),
            pltpu.VMEM((2, K, DD), jnp.float32),
            pltpu.VMEM((2, K, D), jnp.float32),
            pltpu.SemaphoreType.DMA,
            pltpu.SemaphoreType.DMA,
            pltpu.SemaphoreType.DMA,
            pltpu.SemaphoreType.DMA,
            pltpu.SemaphoreType.DMA,
            pltpu.SemaphoreType.DMA,
        ],
    )
    def body(ent_hbm, att_hbm, w_hbm, out_hbm, idx_v, entc_v, rows_v, outc_v,
             semr0, semr1, seme0, seme1, semo0, semo1):
        wid = lax.axis_index("s") * NC + lax.axis_index("c")
        base = wid * BPW
        pltpu.sync_copy(att_hbm.at[wid], idx_v)

        semr = (semr0, semr1)
        seme = (seme0, seme1)
        semo = (semo0, semo1)

        def rows_desc(c):
            p = c % 2
            return pltpu.make_async_copy(w_hbm.at[idx_v.at[c]], rows_v.at[p],
                                         semr[p])

        def ent_desc(c):
            p = c % 2
            return pltpu.make_async_copy(ent_hbm.at[pl.ds(base + c * K, K)],
                                         entc_v.at[p], seme[p])

        def out_desc(c):
            p = c % 2
            return pltpu.make_async_copy(outc_v.at[p],
                                         out_hbm.at[pl.ds(base + c * K, K)],
                                         semo[p])

        rows_desc(0).start()
        ent_desc(0).start()
        for c in range(NCHUNK):
            p = c % 2
            if c + 1 < NCHUNK:
                rows_desc(c + 1).start()
                ent_desc(c + 1).start()
            if c >= 2:
                out_desc(c - 2).wait()
            rows_desc(c).wait()
            ent_desc(c).wait()

            def elem(e, _, p=p):
                ev0 = entc_v[p, e, pl.ds(0, L)]
                ev1 = entc_v[p, e, pl.ds(L, L)]
                a0 = jnp.zeros((L,), jnp.float32)
                a1 = jnp.zeros((L,), jnp.float32)
                b0 = jnp.zeros((L,), jnp.float32)
                b1 = jnp.zeros((L,), jnp.float32)
                for i in range(0, D, 2):
                    ev = ev0 if i < L else ev1
                    s0 = ev[i % L]
                    s1 = ev[(i + 1) % L]
                    a0 = a0 + s0 * rows_v[p, e, pl.ds(i * D, L)]
                    a1 = a1 + s0 * rows_v[p, e, pl.ds(i * D + L, L)]
                    b0 = b0 + s1 * rows_v[p, e, pl.ds(i * D + D, L)]
                    b1 = b1 + s1 * rows_v[p, e, pl.ds(i * D + D + L, L)]
                outc_v[p, e, pl.ds(0, L)] = a0 + b0
                outc_v[p, e, pl.ds(L, L)] = a1 + b1
                return 0

            lax.fori_loop(0, K, elem, 0)
            out_desc(c).start()

        out_desc(NCHUNK - 2).wait()
        out_desc(NCHUNK - 1).wait()

    return body(ent, att3, W_att)


def kernel(ent, att, W_att):
    att3 = att.astype(jnp.int32).reshape(NW, NCHUNK, K)
    return _sc_call(ent, att3, W_att)


# speedup vs baseline: 2.0407x; 1.0069x over previous
import functools

import jax
import jax.numpy as jnp
from jax import lax
from jax.experimental import pallas as pl
from jax.experimental.pallas import tpu as pltpu
from jax.experimental.pallas import tpu_sc as plsc

D = 32
DD = D * D
B = 16384
L = 16
NC, NS = 2, 16
NW = NC * NS
BPW = B // NW
K = 32
NCHUNK = BPW // K


def _sc_call(ent, att3, W_att):
    mesh = plsc.VectorSubcoreMesh(core_axis_name="c", subcore_axis_name="s")

    @functools.partial(
        pl.kernel,
        mesh=mesh,
        out_type=jax.ShapeDtypeStruct((B, D), jnp.float32),
        scratch_types=[
            pltpu.VMEM((NCHUNK, K), jnp.int32),
            pltpu.VMEM((2, K, D), jnp.float32),
            pltpu.VMEM((2, K, DD), jnp.float32),
            pltpu.VMEM((2, K, D), jnp.float32),
            pltpu.SemaphoreType.DMA,
            pltpu.SemaphoreType.DMA,
            pltpu.SemaphoreType.DMA,
            pltpu.SemaphoreType.DMA,
            pltpu.SemaphoreType.DMA,
            pltpu.SemaphoreType.DMA,
        ],
    )
    def body(ent_hbm, att_hbm, w_hbm, out_hbm, idx_v, entc_v, rows_v, outc_v,
             semr0, semr1, seme0, seme1, semo0, semo1):
        wid = lax.axis_index("s") * NC + lax.axis_index("c")
        base = wid * BPW
        pltpu.sync_copy(att_hbm.at[wid], idx_v)

        semr = (semr0, semr1)
        seme = (seme0, seme1)
        semo = (semo0, semo1)

        def rows_desc(c):
            p = c % 2
            return pltpu.make_async_copy(w_hbm.at[idx_v.at[c]], rows_v.at[p],
                                         semr[p])

        def ent_desc(c):
            p = c % 2
            return pltpu.make_async_copy(ent_hbm.at[pl.ds(base + c * K, K)],
                                         entc_v.at[p], seme[p])

        def out_desc(c):
            p = c % 2
            return pltpu.make_async_copy(outc_v.at[p],
                                         out_hbm.at[pl.ds(base + c * K, K)],
                                         semo[p])

        rows_desc(0).start()
        ent_desc(0).start()
        for c in range(NCHUNK):
            p = c % 2
            if c + 1 < NCHUNK:
                rows_desc(c + 1).start()
                ent_desc(c + 1).start()
            if c >= 2:
                out_desc(c - 2).wait()
            rows_desc(c).wait()
            ent_desc(c).wait()

            def elem(e, _, p=p):
                ev0 = entc_v[p, e, pl.ds(0, L)]
                ev1 = entc_v[p, e, pl.ds(L, L)]
                a0 = jnp.zeros((L,), jnp.float32)
                a1 = jnp.zeros((L,), jnp.float32)
                b0 = jnp.zeros((L,), jnp.float32)
                b1 = jnp.zeros((L,), jnp.float32)
                for i in range(0, D, 2):
                    ev = ev0 if i < L else ev1
                    s0 = ev[i % L]
                    s1 = ev[(i + 1) % L]
                    a0 = a0 + s0 * rows_v[p, e, pl.ds(i * D, L)]
                    a1 = a1 + s0 * rows_v[p, e, pl.ds(i * D + L, L)]
                    b0 = b0 + s1 * rows_v[p, e, pl.ds(i * D + D, L)]
                    b1 = b1 + s1 * rows_v[p, e, pl.ds(i * D + D + L, L)]
                outc_v[p, e, pl.ds(0, L)] = a0 + b0
                outc_v[p, e, pl.ds(L, L)] = a1 + b1
                return 0

            lax.fori_loop(0, K, elem, 0, unroll=2)
            out_desc(c).start()

        out_desc(NCHUNK - 2).wait()
        out_desc(NCHUNK - 1).wait()

    return body(ent, att3, W_att)


def kernel(ent, att, W_att):
    att3 = att.astype(jnp.int32).reshape(NW, NCHUNK, K)
    return _sc_call(ent, att3, W_att)


# speedup vs baseline: 2.1981x; 1.0771x over previous
import functools

import jax
import jax.numpy as jnp
from jax import lax
from jax.experimental import pallas as pl
from jax.experimental.pallas import tpu as pltpu
from jax.experimental.pallas import tpu_sc as plsc

D = 32
DD = D * D
B = 16384
L = 16
NC, NS = 2, 16
NW = NC * NS
BPW = B // NW
K = 32
NCHUNK = BPW // K


def _sc_call(ent, att, W_att):
    mesh = plsc.VectorSubcoreMesh(core_axis_name="c", subcore_axis_name="s")

    @functools.partial(
        pl.kernel,
        mesh=mesh,
        out_type=jax.ShapeDtypeStruct((B, D), jnp.float32),
        scratch_types=[
            pltpu.VMEM((BPW,), jnp.int32),
            pltpu.VMEM((2, K, D), jnp.float32),
            pltpu.VMEM((2, K, DD), jnp.float32),
            pltpu.VMEM((2, K, D), jnp.float32),
            pltpu.SemaphoreType.DMA,
            pltpu.SemaphoreType.DMA,
            pltpu.SemaphoreType.DMA,
            pltpu.SemaphoreType.DMA,
            pltpu.SemaphoreType.DMA,
            pltpu.SemaphoreType.DMA,
        ],
    )
    def body(ent_hbm, att_hbm, w_hbm, out_hbm, idx_v, entc_v, rows_v, outc_v,
             semr0, semr1, seme0, seme1, semo0, semo1):
        wid = lax.axis_index("s") * NC + lax.axis_index("c")
        base = wid * BPW
        pltpu.sync_copy(att_hbm.at[pl.ds(base, BPW)], idx_v)

        semr = (semr0, semr1)
        seme = (seme0, seme1)
        semo = (semo0, semo1)

        def rows_desc(c, par):
            return pltpu.make_async_copy(
                w_hbm.at[idx_v.at[pl.ds(c * K, K)]], rows_v.at[par], semr[par])

        def ent_desc(c, par):
            return pltpu.make_async_copy(
                ent_hbm.at[pl.ds(base + c * K, K)], entc_v.at[par], seme[par])

        def out_desc(c, par):
            return pltpu.make_async_copy(
                outc_v.at[par], out_hbm.at[pl.ds(base + c * K, K)], semo[par])

        def compute(par):
            def elem(e, _):
                ev0 = entc_v[par, e, pl.ds(0, L)]
                ev1 = entc_v[par, e, pl.ds(L, L)]
                a0 = jnp.zeros((L,), jnp.float32)
                a1 = jnp.zeros((L,), jnp.float32)
                b0 = jnp.zeros((L,), jnp.float32)
                b1 = jnp.zeros((L,), jnp.float32)
                for i in range(0, D, 2):
                    ev = ev0 if i < L else ev1
                    s0 = ev[i % L]
                    s1 = ev[(i + 1) % L]
                    a0 = a0 + s0 * rows_v[par, e, pl.ds(i * D, L)]
                    a1 = a1 + s0 * rows_v[par, e, pl.ds(i * D + L, L)]
                    b0 = b0 + s1 * rows_v[par, e, pl.ds(i * D + D, L)]
                    b1 = b1 + s1 * rows_v[par, e, pl.ds(i * D + D + L, L)]
                outc_v[par, e, pl.ds(0, L)] = a0 + b0
                outc_v[par, e, pl.ds(L, L)] = a1 + b1
                return 0

            lax.fori_loop(0, K, elem, 0, unroll=2)

        rows_desc(0, 0).start()
        ent_desc(0, 0).start()
        rows_desc(1, 1).start()
        ent_desc(1, 1).start()

        def pair(c2, _):
            for par in range(2):
                c = c2 * 2 + par
                rows_desc(c, par).wait()
                ent_desc(c, par).wait()

                @pl.when(c2 > 0)
                def _():
                    out_desc(c - 2, par).wait()

                compute(par)

                @pl.when(c2 < NCHUNK // 2 - 1)
                def _():
                    rows_desc(c + 2, par).start()
                    ent_desc(c + 2, par).start()

                out_desc(c, par).start()
            return 0

        lax.fori_loop(0, NCHUNK // 2, pair, 0)
        out_desc(NCHUNK - 2, 0).wait()
        out_desc(NCHUNK - 1, 1).wait()

    return body(ent, att, W_att)


def kernel(ent, att, W_att):
    return _sc_call(ent, att.astype(jnp.int32), W_att)


# speedup vs baseline: 2.2023x; 1.0019x over previous
import functools

import jax
import jax.numpy as jnp
from jax import lax
from jax.experimental import pallas as pl
from jax.experimental.pallas import tpu as pltpu
from jax.experimental.pallas import tpu_sc as plsc

D = 32
DD = D * D
B = 16384
L = 16
NC, NS = 2, 16
NW = NC * NS
BPW = B // NW
K = 32
NCHUNK = BPW // K


def _sc_call(ent, att, W_att):
    mesh = plsc.VectorSubcoreMesh(core_axis_name="c", subcore_axis_name="s")

    @functools.partial(
        pl.kernel,
        mesh=mesh,
        out_type=jax.ShapeDtypeStruct((B, 128), jnp.float32),
        scratch_types=[
            pltpu.VMEM((BPW,), jnp.int32),
            pltpu.VMEM((2, K, D), jnp.float32),
            pltpu.VMEM((2, K, DD), jnp.float32),
            pltpu.VMEM((2, K, 128), jnp.float32),
            pltpu.SemaphoreType.DMA,
            pltpu.SemaphoreType.DMA,
            pltpu.SemaphoreType.DMA,
            pltpu.SemaphoreType.DMA,
            pltpu.SemaphoreType.DMA,
            pltpu.SemaphoreType.DMA,
        ],
    )
    def body(ent_hbm, att_hbm, w_hbm, out_hbm, idx_v, entc_v, rows_v, outc_v,
             semr0, semr1, seme0, seme1, semo0, semo1):
        wid = lax.axis_index("s") * NC + lax.axis_index("c")
        base = wid * BPW
        pltpu.sync_copy(att_hbm.at[pl.ds(base, BPW)], idx_v)

        semr = (semr0, semr1)
        seme = (seme0, seme1)
        semo = (semo0, semo1)

        def rows_desc(c, par):
            return pltpu.make_async_copy(
                w_hbm.at[idx_v.at[pl.ds(c * K, K)]], rows_v.at[par], semr[par])

        def ent_desc(c, par):
            return pltpu.make_async_copy(
                ent_hbm.at[pl.ds(base + c * K, K)], entc_v.at[par], seme[par])

        def out_desc(c, par):
            return pltpu.make_async_copy(
                outc_v.at[par], out_hbm.at[pl.ds(base + c * K, K)], semo[par])

        def compute(par):
            def elem(e, _):
                ev0 = entc_v[par, e, pl.ds(0, L)]
                ev1 = entc_v[par, e, pl.ds(L, L)]
                a0 = jnp.zeros((L,), jnp.float32)
                a1 = jnp.zeros((L,), jnp.float32)
                b0 = jnp.zeros((L,), jnp.float32)
                b1 = jnp.zeros((L,), jnp.float32)
                for i in range(0, D, 2):
                    ev = ev0 if i < L else ev1
                    s0 = ev[i % L]
                    s1 = ev[(i + 1) % L]
                    a0 = a0 + s0 * rows_v[par, e, pl.ds(i * D, L)]
                    a1 = a1 + s0 * rows_v[par, e, pl.ds(i * D + L, L)]
                    b0 = b0 + s1 * rows_v[par, e, pl.ds(i * D + D, L)]
                    b1 = b1 + s1 * rows_v[par, e, pl.ds(i * D + D + L, L)]
                outc_v[par, e, pl.ds(0, L)] = a0 + b0
                outc_v[par, e, pl.ds(L, L)] = a1 + b1
                return 0

            lax.fori_loop(0, K, elem, 0, unroll=2)

        rows_desc(0, 0).start()
        ent_desc(0, 0).start()
        rows_desc(1, 1).start()
        ent_desc(1, 1).start()

        def pair(c2, _):
            for par in range(2):
                c = c2 * 2 + par
                rows_desc(c, par).wait()
                ent_desc(c, par).wait()

                @pl.when(c2 > 0)
                def _():
                    out_desc(c - 2, par).wait()

                compute(par)

                @pl.when(c2 < NCHUNK // 2 - 1)
                def _():
                    rows_desc(c + 2, par).start()
                    ent_desc(c + 2, par).start()

                out_desc(c, par).start()
            return 0

        lax.fori_loop(0, NCHUNK // 2, pair, 0)
        out_desc(NCHUNK - 2, 0).wait()
        out_desc(NCHUNK - 1, 1).wait()

    return body(ent, att, W_att)


def kernel(ent, att, W_att):
    out_pad = _sc_call(ent, att.astype(jnp.int32), W_att)
    return jax.lax.slice(out_pad, (0, 0), (B, D))


# speedup vs baseline: 2.2052x; 1.0013x over previous
import functools

import jax
import jax.numpy as jnp
from jax import lax
from jax.experimental import pallas as pl
from jax.experimental.pallas import tpu as pltpu
from jax.experimental.pallas import tpu_sc as plsc

D = 32
DD = D * D
B = 16384
L = 16
NC, NS = 2, 16
NW = NC * NS
BPW = B // NW
K = 32
NCHUNK = BPW // K


def _sc_call(ent, att, W_att):
    mesh = plsc.VectorSubcoreMesh(core_axis_name="c", subcore_axis_name="s")

    @functools.partial(
        pl.kernel,
        mesh=mesh,
        compiler_params=pltpu.CompilerParams(use_tc_tiling_on_sc=True),
        out_type=jax.ShapeDtypeStruct((B, 128), jnp.float32),
        scratch_types=[
            pltpu.VMEM((BPW,), jnp.int32),
            pltpu.VMEM((2, K, D), jnp.float32),
            pltpu.VMEM((2, K, DD), jnp.float32),
            pltpu.VMEM((2, K, 128), jnp.float32),
            pltpu.SemaphoreType.DMA,
            pltpu.SemaphoreType.DMA,
            pltpu.SemaphoreType.DMA,
            pltpu.SemaphoreType.DMA,
            pltpu.SemaphoreType.DMA,
            pltpu.SemaphoreType.DMA,
        ],
    )
    def body(ent_hbm, att_hbm, w_hbm, out_hbm, idx_v, entc_v, rows_v, outc_v,
             semr0, semr1, seme0, seme1, semo0, semo1):
        wid = lax.axis_index("s") * NC + lax.axis_index("c")
        base = wid * BPW
        pltpu.sync_copy(att_hbm.at[pl.ds(base, BPW)], idx_v)

        semr = (semr0, semr1)
        seme = (seme0, seme1)
        semo = (semo0, semo1)

        def rows_desc(c, par):
            return pltpu.make_async_copy(
                w_hbm.at[idx_v.at[pl.ds(c * K, K)]], rows_v.at[par], semr[par])

        def ent_desc(c, par):
            return pltpu.make_async_copy(
                ent_hbm.at[pl.ds(base + c * K, K)], entc_v.at[par], seme[par])

        def out_desc(c, par):
            return pltpu.make_async_copy(
                outc_v.at[par], out_hbm.at[pl.ds(base + c * K, K)], semo[par])

        def compute(par):
            def elem(e, _):
                ev0 = entc_v[par, e, pl.ds(0, L)]
                ev1 = entc_v[par, e, pl.ds(L, L)]
                a0 = jnp.zeros((L,), jnp.float32)
                a1 = jnp.zeros((L,), jnp.float32)
                b0 = jnp.zeros((L,), jnp.float32)
                b1 = jnp.zeros((L,), jnp.float32)
                for i in range(0, D, 2):
                    ev = ev0 if i < L else ev1
                    s0 = ev[i % L]
                    s1 = ev[(i + 1) % L]
                    a0 = a0 + s0 * rows_v[par, e, pl.ds(i * D, L)]
                    a1 = a1 + s0 * rows_v[par, e, pl.ds(i * D + L, L)]
                    b0 = b0 + s1 * rows_v[par, e, pl.ds(i * D + D, L)]
                    b1 = b1 + s1 * rows_v[par, e, pl.ds(i * D + D + L, L)]
                outc_v[par, e, pl.ds(0, L)] = a0 + b0
                outc_v[par, e, pl.ds(L, L)] = a1 + b1
                return 0

            lax.fori_loop(0, K, elem, 0, unroll=2)

        rows_desc(0, 0).start()
        ent_desc(0, 0).start()
        rows_desc(1, 1).start()
        ent_desc(1, 1).start()

        def pair(c2, _):
            for par in range(2):
                c = c2 * 2 + par
                rows_desc(c, par).wait()
                ent_desc(c, par).wait()

                @pl.when(c2 > 0)
                def _():
                    out_desc(c - 2, par).wait()

                compute(par)

                @pl.when(c2 < NCHUNK // 2 - 1)
                def _():
                    rows_desc(c + 2, par).start()
                    ent_desc(c + 2, par).start()

                out_desc(c, par).start()
            return 0

        lax.fori_loop(0, NCHUNK // 2, pair, 0)
        out_desc(NCHUNK - 2, 0).wait()
        out_desc(NCHUNK - 1, 1).wait()

    return body(ent, att, W_att)


def kernel(ent, att, W_att):
    out_pad = _sc_call(ent, att.astype(jnp.int32), W_att)
    return jax.lax.slice(out_pad, (0, 0), (B, D))
